# Initial kernel scaffold; baseline (speedup 1.0000x reference)
import functools

import jax
import jax.numpy as jnp
from jax import lax
from jax.experimental import pallas as pl
from jax.experimental.pallas import tpu as pltpu
from jax.experimental.pallas import tpu_sc as plsc

N_NODES = 10000
N_EDGES = 320000
D_FEAT = 128

NUM_CORES = 2
NUM_SUBCORES = 16
NUM_WORKERS = NUM_CORES * NUM_SUBCORES
EDGES_PER_WORKER = N_EDGES // NUM_WORKERS
CHUNK = 80
NUM_CHUNKS = EDGES_PER_WORKER // CHUNK
ROWS_PER_TILE = N_NODES // NUM_SUBCORES
ZROWS = 125


def _sc_spmm(dst_hbm, src_hbm, val_hbm, emb_hbm, out_hbm,
             idxs_v, idxd_v, vals_v, rows_v, zbuf_v, acc_sh, sem):
    c = lax.axis_index("c")
    s = lax.axis_index("s")
    wid = c * NUM_SUBCORES + s

    def zero_row(i, carry):
        z = jnp.zeros((16,), jnp.float32)
        for j in range(D_FEAT // 16):
            zbuf_v[i, pl.ds(j * 16, 16)] = z
        return carry

    lax.fori_loop(0, ZROWS, zero_row, 0)
    for j in range(ROWS_PER_TILE // ZROWS):
        pltpu.sync_copy(zbuf_v, acc_sh.at[pl.ds(s * ROWS_PER_TILE + j * ZROWS, ZROWS)])
    plsc.subcore_barrier()

    ebase = wid * EDGES_PER_WORKER

    def chunk_body(ci, carry):
        off = ebase + ci * CHUNK
        pltpu.sync_copy(src_hbm.at[pl.ds(off, CHUNK)], idxs_v)
        pltpu.sync_copy(dst_hbm.at[pl.ds(off, CHUNK)], idxd_v)
        pltpu.sync_copy(val_hbm.at[pl.ds(off, CHUNK)], vals_v)
        pltpu.async_copy(emb_hbm.at[idxs_v], rows_v, sem).wait()

        def scale_row(e, carry2):
            v = vals_v[e]
            for j in range(D_FEAT // 16):
                sl = pl.ds(j * 16, 16)
                rows_v[e, sl] = rows_v[e, sl] * v
            return carry2

        lax.fori_loop(0, CHUNK, scale_row, 0)
        pltpu.sync_copy(rows_v, acc_sh.at[idxd_v], add=True)
        return carry

    lax.fori_loop(0, NUM_CHUNKS, chunk_body, 0)
    plsc.subcore_barrier()

    for j in range(ROWS_PER_TILE // ZROWS):
        row0 = s * ROWS_PER_TILE + j * ZROWS
        pltpu.sync_copy(acc_sh.at[pl.ds(row0, ZROWS)],
                        out_hbm.at[c, pl.ds(row0, ZROWS)])


def _tc_add(a_ref, b_ref, o_ref):
    o_ref[...] = a_ref[...] + b_ref[...]


def kernel(edge_index, edge_values, embeds):
    dst = edge_index[0].astype(jnp.int32)
    src = edge_index[1].astype(jnp.int32)
    val = edge_values.astype(jnp.float32)

    mesh = plsc.VectorSubcoreMesh(core_axis_name="c", subcore_axis_name="s")
    partials = pl.kernel(
        _sc_spmm,
        mesh=mesh,
        out_type=jax.ShapeDtypeStruct((NUM_CORES, N_NODES, D_FEAT), jnp.float32),
        scratch_types=[
            pltpu.VMEM((CHUNK,), jnp.int32),
            pltpu.VMEM((CHUNK,), jnp.int32),
            pltpu.VMEM((CHUNK,), jnp.float32),
            pltpu.VMEM((CHUNK, D_FEAT), jnp.float32),
            pltpu.VMEM((ZROWS, D_FEAT), jnp.float32),
            pltpu.VMEM_SHARED((N_NODES, D_FEAT), jnp.float32),
            pltpu.SemaphoreType.DMA,
        ],
    )(dst, src, val, embeds)

    rows_blk = 1000
    out = pl.pallas_call(
        _tc_add,
        grid=(N_NODES // rows_blk,),
        in_specs=[
            pl.BlockSpec((rows_blk, D_FEAT), lambda i: (i, 0)),
            pl.BlockSpec((rows_blk, D_FEAT), lambda i: (i, 0)),
        ],
        out_specs=pl.BlockSpec((rows_blk, D_FEAT), lambda i: (i, 0)),
        out_shape=jax.ShapeDtypeStruct((N_NODES, D_FEAT), jnp.float32),
    )(partials[0], partials[1])
    return out


# speedup vs baseline: 4.2767x; 4.2767x over previous
import functools

import jax
import jax.numpy as jnp
from jax import lax
from jax.experimental import pallas as pl
from jax.experimental.pallas import tpu as pltpu
from jax.experimental.pallas import tpu_sc as plsc

N_NODES = 10000
N_EDGES = 320000
D_FEAT = 128

NUM_CORES = 2
NUM_SUBCORES = 16
NUM_WORKERS = NUM_CORES * NUM_SUBCORES
EDGES_PER_WORKER = N_EDGES // NUM_WORKERS
CHUNK = 80
NUM_CHUNKS = EDGES_PER_WORKER // CHUNK
ROW_BLK = 16
NUM_ROW_BLKS = N_NODES // ROW_BLK


def _sc_spmm(dst_hbm, src_hbm, val_hbm, emb_hbm, out_hbm,
             idxs_v, idxd_v, vals_v, rows_v, zbuf_v, acc_sh, sem):
    c = lax.axis_index("c")
    s = lax.axis_index("s")
    wid = c * NUM_SUBCORES + s

    z = jnp.zeros((16,), jnp.float32)
    for i in range(ROW_BLK):
        for j in range(D_FEAT // 16):
            zbuf_v[i, pl.ds(j * 16, 16)] = z
    b0 = (s * NUM_ROW_BLKS) // NUM_SUBCORES
    b1 = ((s + 1) * NUM_ROW_BLKS) // NUM_SUBCORES

    def zero_blk(b, carry):
        row0 = pl.multiple_of(b * ROW_BLK, ROW_BLK)
        pltpu.sync_copy(zbuf_v, acc_sh.at[pl.ds(row0, ROW_BLK)])
        return carry

    lax.fori_loop(b0, b1, zero_blk, 0)
    plsc.subcore_barrier()

    ebase = wid * EDGES_PER_WORKER

    def chunk_body(ci, carry):
        off = ebase + ci * CHUNK
        pltpu.sync_copy(src_hbm.at[pl.ds(off, CHUNK)], idxs_v)
        pltpu.sync_copy(dst_hbm.at[pl.ds(off, CHUNK)], idxd_v)
        pltpu.sync_copy(val_hbm.at[pl.ds(off, CHUNK)], vals_v)
        pltpu.async_copy(emb_hbm.at[idxs_v], rows_v, sem).wait()

        def scale_group(g, carry2):
            vv = vals_v[pl.ds(g * 16, 16)]
            for i in range(16):
                v = vv[i]
                e = g * 16 + i
                for j in range(D_FEAT // 16):
                    sl = pl.ds(j * 16, 16)
                    rows_v[e, sl] = rows_v[e, sl] * v
            return carry2

        lax.fori_loop(0, CHUNK // 16, scale_group, 0)
        pltpu.sync_copy(rows_v, acc_sh.at[idxd_v], add=True)
        return carry

    lax.fori_loop(0, NUM_CHUNKS, chunk_body, 0)
    plsc.subcore_barrier()

    def drain_blk(b, carry):
        row0 = pl.multiple_of(b * ROW_BLK, ROW_BLK)
        pltpu.sync_copy(acc_sh.at[pl.ds(row0, ROW_BLK)],
                        out_hbm.at[c, pl.ds(row0, ROW_BLK)])
        return carry

    lax.fori_loop(b0, b1, drain_blk, 0)


def _tc_add(a_ref, b_ref, o_ref):
    o_ref[...] = a_ref[...] + b_ref[...]


def kernel(edge_index, edge_values, embeds):
    dst = edge_index[0].astype(jnp.int32)
    src = edge_index[1].astype(jnp.int32)
    val = edge_values.astype(jnp.float32)

    mesh = plsc.VectorSubcoreMesh(core_axis_name="c", subcore_axis_name="s")
    partials = pl.kernel(
        _sc_spmm,
        mesh=mesh,
        out_type=jax.ShapeDtypeStruct((NUM_CORES, N_NODES, D_FEAT), jnp.float32),
        scratch_types=[
            pltpu.VMEM((CHUNK,), jnp.int32),
            pltpu.VMEM((CHUNK,), jnp.int32),
            pltpu.VMEM((CHUNK,), jnp.float32),
            pltpu.VMEM((CHUNK, D_FEAT), jnp.float32),
            pltpu.VMEM((ROW_BLK, D_FEAT), jnp.float32),
            pltpu.VMEM_SHARED((N_NODES, D_FEAT), jnp.float32),
            pltpu.SemaphoreType.DMA,
        ],
    )(dst, src, val, embeds)

    rows_blk = 1000
    out = pl.pallas_call(
        _tc_add,
        grid=(N_NODES // rows_blk,),
        in_specs=[
            pl.BlockSpec((rows_blk, D_FEAT), lambda i: (i, 0)),
            pl.BlockSpec((rows_blk, D_FEAT), lambda i: (i, 0)),
        ],
        out_specs=pl.BlockSpec((rows_blk, D_FEAT), lambda i: (i, 0)),
        out_shape=jax.ShapeDtypeStruct((N_NODES, D_FEAT), jnp.float32),
    )(partials[0], partials[1])
    return out
